# baseline (speedup 1.0000x reference)
import jax
import jax.numpy as jnp
from jax.experimental import pallas as pl
from jax.experimental.pallas import tpu as pltpu

_B = 2
_S = 16
_I = 4
_FIN = 8
_H = 32
_HEADS = 4
_G = 4
_ROWS = _G * _B * _S
_PROWS = _G * _B * _I
_CH = 4
_TROWS = _CH * _ROWS
_TPROWS = _CH * _PROWS
_LANES = 128

_R_W1 = 0
_R_W2 = 8
_R_AD1 = 40
_R_AS1 = 48
_R_AD2 = 56
_R_AS2 = 64
_R_GB = 72
_R_WH1 = 80
_R_WH1B = 112
_R_WH2 = 120
_R_WH2B = 128
_WBLK = 136

_C_SAME = 0
_C_PSAME = 128
_C_GRP = 160
_C_TILE = 288
_C_SEL = 304
_C_CHSEL = 320
_CROWS = 832


def _make_const_slab():
    f32 = jnp.float32
    idx = jnp.arange(_ROWS)
    same = (idx[:, None] // _S == idx[None, :] // _S).astype(f32)
    pidx = jnp.arange(_PROWS)
    psame = (pidx[:, None] // _I == idx[None, :] // _S).astype(f32)
    tp = jnp.arange(_TPROWS)
    grp = (tp[:, None] // _I == tp[None, :] // _I).astype(f32)
    tile = (jnp.arange(_LANES)[None, :] % _S == jnp.arange(_S)[:, None]).astype(f32)
    slab = jnp.zeros((_CROWS, _LANES), f32)
    slab = slab.at[_C_SAME:_C_SAME + _ROWS, :].set(same)
    slab = slab.at[_C_PSAME:_C_PSAME + _PROWS, :].set(psame)
    slab = slab.at[_C_GRP:_C_GRP + _TPROWS, :].set(grp)
    slab = slab.at[_C_TILE:_C_TILE + _S, :].set(tile)
    sel_rows = jnp.zeros((_HEADS * _HEADS, _LANES), f32)
    for h in range(_HEADS):
        sel_rows = sel_rows.at[h * _HEADS + h, :].set(1.0)
    slab = slab.at[_C_SEL:_C_SEL + _HEADS * _HEADS, :].set(sel_rows)
    chsel = (jnp.arange(_TROWS)[:, None] // _ROWS
             == jnp.arange(_CH)[None, :]).astype(f32)
    slab = slab.at[_C_CHSEL:_C_CHSEL + _TROWS, 0:_CH].set(chsel)
    return slab


def _gat_layer(feat, w_all, a_dst_rows, a_src_rows, bias, mask, sel, ones_h, chsel):
    f32 = jnp.float32
    hf = jnp.dot(feat, w_all, preferred_element_type=f32)
    ed = jax.lax.dot_general(hf, a_dst_rows,
                             dimension_numbers=(((1,), (1,)), ((), ())),
                             preferred_element_type=f32)
    esl = jax.lax.dot_general(a_src_rows, hf,
                              dimension_numbers=(((1,), (1,)), ((), ())),
                              preferred_element_type=f32)

    aaug = jnp.concatenate([ed, chsel], axis=1)
    acc = None
    for h in range(_HEADS):
        b_h = jnp.concatenate(
            [sel[h], esl[h:h + 1, :].reshape(_CH, _ROWS)], axis=0)
        t = jnp.dot(aaug, b_h, preferred_element_type=f32)
        e = jnp.maximum(t, 0.2 * t) + mask
        p = jnp.exp(e)
        dn = jnp.dot(p, ones_h, preferred_element_type=f32)
        part = jnp.concatenate(
            [jnp.dot(p[c * _ROWS:(c + 1) * _ROWS, :],
                     hf[c * _ROWS:(c + 1) * _ROWS, :],
                     preferred_element_type=f32)[:, h * _H:(h + 1) * _H]
             for c in range(_CH)], axis=0)
        part = part / dn
        acc = part if acc is None else acc + part
    return acc * (1.0 / _HEADS) + bias


def _fused_kernel(x_ref, adj_ref, memb_ref, w_ref, c_ref, out_ref):
    f32 = jnp.float32
    x = x_ref[...]
    adjc = adj_ref[...]
    membc = memb_ref[...]

    w1 = w_ref[_R_W1:_R_W1 + _FIN, :]
    w2 = w_ref[_R_W2:_R_W2 + _H, :]
    ad1 = w_ref[_R_AD1:_R_AD1 + _HEADS, :]
    as1 = w_ref[_R_AS1:_R_AS1 + _HEADS, :]
    ad2 = w_ref[_R_AD2:_R_AD2 + _HEADS, :]
    as2 = w_ref[_R_AS2:_R_AS2 + _HEADS, :]
    b1 = w_ref[_R_GB:_R_GB + 1, 0:_H]
    b2 = w_ref[_R_GB + 1:_R_GB + 2, 0:_H]
    wh1_w = w_ref[_R_WH1:_R_WH1 + _H, 0:2 * _H]
    wh1_b = w_ref[_R_WH1B:_R_WH1B + 1, 0:2 * _H]
    wh2_wt = w_ref[_R_WH2:_R_WH2 + 2, 0:2 * _H]
    wh2_b = w_ref[_R_WH2B:_R_WH2B + 1, 0:2]

    same1 = c_ref[_C_SAME:_C_SAME + _ROWS, :]
    psame1 = c_ref[_C_PSAME:_C_PSAME + _PROWS, :]
    grp = c_ref[_C_GRP:_C_GRP + _TPROWS, :]
    tile = c_ref[_C_TILE:_C_TILE + _S, :]
    sel = [c_ref[_C_SEL + h * _HEADS:_C_SEL + h * _HEADS + _HEADS, :]
           for h in range(_HEADS)]
    chsel = c_ref[_C_CHSEL:_C_CHSEL + _TROWS, 0:_CH]
    same = jnp.concatenate([same1] * _CH, axis=0)
    psame = jnp.concatenate([psame1] * _CH, axis=0)
    ones_h = jnp.ones((_ROWS, _H), f32)
    ones_sh = jnp.ones((_S, _H), f32)

    adj_t = jnp.dot(adjc, tile, preferred_element_type=f32)
    mask = jnp.where(adj_t * same > 0.5, 0.0, -1e9)

    h1 = jnp.maximum(_gat_layer(x, w1, ad1, as1, b1, mask, sel, ones_h, chsel), 0.0)
    h2 = jnp.maximum(_gat_layer(h1, w2, ad2, as2, b2, mask, sel, ones_h, chsel), 0.0)

    memb_t = jnp.dot(membc, tile, preferred_element_type=f32)
    memb_bd = memb_t * psame
    cntb = jnp.maximum(jnp.dot(membc, ones_sh, preferred_element_type=f32), 1.0)
    sp = jnp.concatenate(
        [jnp.dot(memb_bd[c * _PROWS:(c + 1) * _PROWS, :],
                 h2[c * _ROWS:(c + 1) * _ROWS, :], preferred_element_type=f32)
         for c in range(_CH)], axis=0) / cntb

    hid = jnp.maximum(jnp.dot(sp, wh1_w, preferred_element_type=f32) + wh1_b, 0.0)
    lv = jax.lax.dot_general(hid, wh2_wt,
                             dimension_numbers=(((1,), (1,)), ((), ())),
                             preferred_element_type=f32) + wh2_b

    logits_c = lv[:, 0:1]
    values_c = lv[:, 1:2]
    ez = jnp.exp(logits_c)
    ssum = jnp.dot(grp, ez, preferred_element_type=f32)
    logp = logits_c - jnp.log(ssum)
    pr = jnp.exp(logp)
    ent = -jnp.dot(grp, pr * logp, preferred_element_type=f32)

    lane = jax.lax.broadcasted_iota(jnp.int32, (_TPROWS, 8), 1)
    out = (jnp.where(lane == 0, logits_c, 0.0)
           + jnp.where(lane == 1, values_c, 0.0)
           + jnp.where(lane == 2, logp, 0.0)
           + jnp.where(lane == 3, ent, 0.0))
    out_ref[...] = out.astype(out_ref.dtype)


def kernel(x, adj, memb, w_slab, sample_key):
    n = x.shape[0]
    pairs_per_step = _G * _CH
    nsteps = -(-n // pairs_per_step)
    pad = nsteps * pairs_per_step - n
    if pad:
        x = jnp.concatenate([x, jnp.zeros((pad,) + x.shape[1:], x.dtype)], 0)
        eye = jnp.broadcast_to(jnp.eye(_S, dtype=adj.dtype), (pad, _B, _S, _S))
        adj = jnp.concatenate([adj, eye], 0)
        memb = jnp.concatenate([memb, jnp.zeros((pad,) + memb.shape[1:], memb.dtype)], 0)
    np_ = nsteps * pairs_per_step

    xs = x.reshape(np_ * _B * _S, _FIN)
    adjc = adj.reshape(np_ * _B * _S, _S)
    membc = memb.reshape(np_ * _B * _I, _S)
    cslab = _make_const_slab()

    out = pl.pallas_call(
        _fused_kernel,
        out_shape=jax.ShapeDtypeStruct((np_ * _B * _I, 8), jnp.float32),
        grid=(nsteps,),
        in_specs=[
            pl.BlockSpec((_TROWS, _FIN), lambda g: (g, 0)),
            pl.BlockSpec((_TROWS, _S), lambda g: (g, 0)),
            pl.BlockSpec((_TPROWS, _S), lambda g: (g, 0)),
            pl.BlockSpec((_WBLK, _LANES), lambda g: (0, 0)),
            pl.BlockSpec((_CROWS, _LANES), lambda g: (0, 0)),
        ],
        out_specs=pl.BlockSpec((_TPROWS, 8), lambda g: (g, 0)),
        compiler_params=pltpu.CompilerParams(dimension_semantics=("parallel",)),
        cost_estimate=pl.CostEstimate(flops=int(n * 4.2e6),
                                      transcendentals=int(n * 140_000),
                                      bytes_accessed=int(n * 4200)),
    )(xs, adjc, membc, w_slab, cslab)

    r = out.reshape(np_, _B, _I, 8)[:n]
    logits = r[:, :, :, 0]
    values = r[:, :, :, 1]
    logp_all = r[:, :, :, 2]
    entropy = r[:, :, 0, 3]

    base = jax.random.key(sample_key[0])
    keys = jax.random.split(base, n)
    actions = jax.vmap(lambda ki, lg: jax.random.categorical(ki, lg, axis=-1))(keys, logits)
    log_probs = jnp.take_along_axis(logp_all, actions[:, :, None], axis=-1)[:, :, 0]
    return actions, log_probs, entropy, values


# speedup vs baseline: 1.0581x; 1.0581x over previous
import jax
import jax.numpy as jnp
from jax.experimental import pallas as pl
from jax.experimental.pallas import tpu as pltpu

_B = 2
_S = 16
_I = 4
_FIN = 8
_H = 32
_HEADS = 4
_G = 4
_ROWS = _G * _B * _S
_PROWS = _G * _B * _I
_CH = 4
_TROWS = _CH * _ROWS
_TPROWS = _CH * _PROWS
_LANES = 128

_R_W1 = 0
_R_W2 = 8
_R_AD1 = 40
_R_AS1 = 48
_R_AD2 = 56
_R_AS2 = 64
_R_GB = 72
_R_WH1 = 80
_R_WH1B = 112
_R_WH2 = 120
_R_WH2B = 128
_WBLK = 136

_C_SAME = 0
_C_PSAME = 128
_C_GRP = 160
_C_TILE = 288
_C_SEL = 304
_C_CHSEL = 320
_CROWS = 832


def _make_const_slab():
    f32 = jnp.float32
    idx = jnp.arange(_ROWS)
    same = (idx[:, None] // _S == idx[None, :] // _S).astype(f32)
    pidx = jnp.arange(_PROWS)
    psame = (pidx[:, None] // _I == idx[None, :] // _S).astype(f32)
    tp = jnp.arange(_TPROWS)
    grp = (tp[:, None] // _I == tp[None, :] // _I).astype(f32)
    tile = (jnp.arange(_LANES)[None, :] % _S == jnp.arange(_S)[:, None]).astype(f32)
    slab = jnp.zeros((_CROWS, _LANES), f32)
    slab = slab.at[_C_SAME:_C_SAME + _ROWS, :].set(same)
    slab = slab.at[_C_PSAME:_C_PSAME + _PROWS, :].set(psame)
    slab = slab.at[_C_GRP:_C_GRP + _TPROWS, :].set(grp)
    slab = slab.at[_C_TILE:_C_TILE + _S, :].set(tile)
    sel_rows = jnp.zeros((_HEADS * _HEADS, _LANES), f32)
    for h in range(_HEADS):
        sel_rows = sel_rows.at[h * _HEADS + h, :].set(1.0)
    slab = slab.at[_C_SEL:_C_SEL + _HEADS * _HEADS, :].set(sel_rows)
    chsel = (jnp.arange(_TROWS)[:, None] // _ROWS
             == jnp.arange(_CH)[None, :]).astype(f32)
    slab = slab.at[_C_CHSEL:_C_CHSEL + _TROWS, 0:_CH].set(chsel)
    return slab


def _gat_layer(feat, w_all, a_dst_rows, a_src_rows, bias, mask, sel, ones_h, chsel):
    f32 = jnp.float32
    hf = jnp.dot(feat, w_all, preferred_element_type=f32)
    ed = jax.lax.dot_general(hf, a_dst_rows,
                             dimension_numbers=(((1,), (1,)), ((), ())),
                             preferred_element_type=f32)
    esl = jax.lax.dot_general(a_src_rows, hf,
                              dimension_numbers=(((1,), (1,)), ((), ())),
                              preferred_element_type=f32)

    acc = None
    for h in range(_HEADS):
        s = jnp.dot(ed, sel[h], preferred_element_type=f32)
        esb = jnp.concatenate(
            [jnp.broadcast_to(esl[h:h + 1, c * _ROWS:(c + 1) * _ROWS],
                              (_ROWS, _ROWS)) for c in range(_CH)], axis=0)
        t = s + esb
        e = jnp.maximum(t, 0.2 * t) + mask
        p = jnp.exp(e)
        dn = jnp.dot(p, ones_h, preferred_element_type=f32)
        part = jnp.concatenate(
            [jnp.dot(p[c * _ROWS:(c + 1) * _ROWS, :],
                     hf[c * _ROWS:(c + 1) * _ROWS, :],
                     preferred_element_type=f32)[:, h * _H:(h + 1) * _H]
             for c in range(_CH)], axis=0)
        part = part / dn
        acc = part if acc is None else acc + part
    return acc * (1.0 / _HEADS) + bias


def _fused_kernel(x_ref, adj_ref, memb_ref, w_ref, c_ref, out_ref):
    f32 = jnp.float32
    x = x_ref[...]
    adjc = adj_ref[...]
    membc = memb_ref[...]

    w1 = w_ref[_R_W1:_R_W1 + _FIN, :]
    w2 = w_ref[_R_W2:_R_W2 + _H, :]
    ad1 = w_ref[_R_AD1:_R_AD1 + _HEADS, :]
    as1 = w_ref[_R_AS1:_R_AS1 + _HEADS, :]
    ad2 = w_ref[_R_AD2:_R_AD2 + _HEADS, :]
    as2 = w_ref[_R_AS2:_R_AS2 + _HEADS, :]
    b1 = w_ref[_R_GB:_R_GB + 1, 0:_H]
    b2 = w_ref[_R_GB + 1:_R_GB + 2, 0:_H]
    wh1_w = w_ref[_R_WH1:_R_WH1 + _H, 0:2 * _H]
    wh1_b = w_ref[_R_WH1B:_R_WH1B + 1, 0:2 * _H]
    wh2_wt = w_ref[_R_WH2:_R_WH2 + 2, 0:2 * _H]
    wh2_b = w_ref[_R_WH2B:_R_WH2B + 1, 0:2]

    same1 = c_ref[_C_SAME:_C_SAME + _ROWS, :]
    psame1 = c_ref[_C_PSAME:_C_PSAME + _PROWS, :]
    grp = c_ref[_C_GRP:_C_GRP + _TPROWS, :]
    tile = c_ref[_C_TILE:_C_TILE + _S, :]
    sel = [c_ref[_C_SEL + h * _HEADS:_C_SEL + h * _HEADS + _HEADS, :]
           for h in range(_HEADS)]
    chsel = c_ref[_C_CHSEL:_C_CHSEL + _TROWS, 0:_CH]
    same = jnp.concatenate([same1] * _CH, axis=0)
    psame = jnp.concatenate([psame1] * _CH, axis=0)
    ones_h = jnp.ones((_ROWS, _H), f32)
    ones_sh = jnp.ones((_S, _H), f32)

    adj_t = jnp.dot(adjc, tile, preferred_element_type=f32)
    mask = jnp.where(adj_t * same > 0.5, 0.0, -1e9)

    h1 = jnp.maximum(_gat_layer(x, w1, ad1, as1, b1, mask, sel, ones_h, chsel), 0.0)
    h2 = jnp.maximum(_gat_layer(h1, w2, ad2, as2, b2, mask, sel, ones_h, chsel), 0.0)

    memb_t = jnp.dot(membc, tile, preferred_element_type=f32)
    memb_bd = memb_t * psame
    cntb = jnp.maximum(jnp.dot(membc, ones_sh, preferred_element_type=f32), 1.0)
    sp = jnp.concatenate(
        [jnp.dot(memb_bd[c * _PROWS:(c + 1) * _PROWS, :],
                 h2[c * _ROWS:(c + 1) * _ROWS, :], preferred_element_type=f32)
         for c in range(_CH)], axis=0) / cntb

    hid = jnp.maximum(jnp.dot(sp, wh1_w, preferred_element_type=f32) + wh1_b, 0.0)
    lv = jax.lax.dot_general(hid, wh2_wt,
                             dimension_numbers=(((1,), (1,)), ((), ())),
                             preferred_element_type=f32) + wh2_b

    logits_c = lv[:, 0:1]
    values_c = lv[:, 1:2]
    ez = jnp.exp(logits_c)
    ssum = jnp.dot(grp, ez, preferred_element_type=f32)
    logp = logits_c - jnp.log(ssum)
    pr = jnp.exp(logp)
    ent = -jnp.dot(grp, pr * logp, preferred_element_type=f32)

    lane = jax.lax.broadcasted_iota(jnp.int32, (_TPROWS, 8), 1)
    out = (jnp.where(lane == 0, logits_c, 0.0)
           + jnp.where(lane == 1, values_c, 0.0)
           + jnp.where(lane == 2, logp, 0.0)
           + jnp.where(lane == 3, ent, 0.0))
    out_ref[...] = out.astype(out_ref.dtype)


def kernel(x, adj, memb, w_slab, sample_key):
    n = x.shape[0]
    pairs_per_step = _G * _CH
    nsteps = -(-n // pairs_per_step)
    pad = nsteps * pairs_per_step - n
    if pad:
        x = jnp.concatenate([x, jnp.zeros((pad,) + x.shape[1:], x.dtype)], 0)
        eye = jnp.broadcast_to(jnp.eye(_S, dtype=adj.dtype), (pad, _B, _S, _S))
        adj = jnp.concatenate([adj, eye], 0)
        memb = jnp.concatenate([memb, jnp.zeros((pad,) + memb.shape[1:], memb.dtype)], 0)
    np_ = nsteps * pairs_per_step

    xs = x.reshape(np_ * _B * _S, _FIN)
    adjc = adj.reshape(np_ * _B * _S, _S)
    membc = memb.reshape(np_ * _B * _I, _S)
    cslab = _make_const_slab()

    out = pl.pallas_call(
        _fused_kernel,
        out_shape=jax.ShapeDtypeStruct((np_ * _B * _I, 8), jnp.float32),
        grid=(nsteps,),
        in_specs=[
            pl.BlockSpec((_TROWS, _FIN), lambda g: (g, 0)),
            pl.BlockSpec((_TROWS, _S), lambda g: (g, 0)),
            pl.BlockSpec((_TPROWS, _S), lambda g: (g, 0)),
            pl.BlockSpec((_WBLK, _LANES), lambda g: (0, 0)),
            pl.BlockSpec((_CROWS, _LANES), lambda g: (0, 0)),
        ],
        out_specs=pl.BlockSpec((_TPROWS, 8), lambda g: (g, 0)),
        compiler_params=pltpu.CompilerParams(dimension_semantics=("parallel",)),
        cost_estimate=pl.CostEstimate(flops=int(n * 4.2e6),
                                      transcendentals=int(n * 140_000),
                                      bytes_accessed=int(n * 4200)),
    )(xs, adjc, membc, w_slab, cslab)

    r = out.reshape(np_, _B, _I, 8)[:n]
    logits = r[:, :, :, 0]
    values = r[:, :, :, 1]
    logp_all = r[:, :, :, 2]
    entropy = r[:, :, 0, 3]

    base = jax.random.key(sample_key[0])
    keys = jax.random.split(base, n)
    actions = jax.vmap(lambda ki, lg: jax.random.categorical(ki, lg, axis=-1))(keys, logits)
    log_probs = jnp.take_along_axis(logp_all, actions[:, :, None], axis=-1)[:, :, 0]
    return actions, log_probs, entropy, values


# speedup vs baseline: 1.4524x; 1.3727x over previous
import jax
import jax.numpy as jnp
from jax.experimental import pallas as pl
from jax.experimental.pallas import tpu as pltpu

_B = 2
_S = 16
_I = 4
_FIN = 8
_H = 32
_HEADS = 4
_G = 4
_ROWS = _G * _B * _S
_PROWS = _G * _B * _I
_CH = 8
_TROWS = _CH * _ROWS
_TPROWS = _CH * _PROWS
_LANES = 128

_R_W1 = 0
_R_W2 = 8
_R_AD1 = 40
_R_AS1 = 48
_R_AD2 = 56
_R_AS2 = 64
_R_GB = 72
_R_WH1 = 80
_R_WH1B = 112
_R_WH2 = 120
_R_WH2B = 128
_WBLK = 136

_C_SAME = 0
_C_PSAME = 128
_C_GRP = 160
_C_TILE = 288
_C_SEL = 304
_CROWS = 320


def _make_const_slab():
    f32 = jnp.float32
    idx = jnp.arange(_ROWS)
    same = (idx[:, None] // _S == idx[None, :] // _S).astype(f32)
    pidx = jnp.arange(_PROWS)
    psame = (pidx[:, None] // _I == idx[None, :] // _S).astype(f32)
    tp = jnp.arange(_LANES)
    grp = (tp[:, None] // _I == tp[None, :] // _I).astype(f32)
    tile = (jnp.arange(_LANES)[None, :] % _S == jnp.arange(_S)[:, None]).astype(f32)
    slab = jnp.zeros((_CROWS, _LANES), f32)
    slab = slab.at[_C_SAME:_C_SAME + _ROWS, :].set(same)
    slab = slab.at[_C_PSAME:_C_PSAME + _PROWS, :].set(psame)
    slab = slab.at[_C_GRP:_C_GRP + _LANES, :].set(grp)
    slab = slab.at[_C_TILE:_C_TILE + _S, :].set(tile)
    sel_rows = jnp.zeros((_HEADS * _HEADS, _LANES), f32)
    for h in range(_HEADS):
        sel_rows = sel_rows.at[h * _HEADS + h, :].set(1.0)
    slab = slab.at[_C_SEL:_C_SEL + _HEADS * _HEADS, :].set(sel_rows)
    return slab


def _gat_layer(feat, w_all, a_dst_rows, a_src_rows, bias, mask, sel, ones_h):
    f32 = jnp.float32
    hf = jnp.dot(feat, w_all, preferred_element_type=f32)
    ed = jax.lax.dot_general(hf, a_dst_rows,
                             dimension_numbers=(((1,), (1,)), ((), ())),
                             preferred_element_type=f32)
    esl = jax.lax.dot_general(a_src_rows, hf,
                              dimension_numbers=(((1,), (1,)), ((), ())),
                              preferred_element_type=f32)

    acc = None
    for h in range(_HEADS):
        s = jnp.dot(ed, sel[h], preferred_element_type=f32)
        esb = jnp.concatenate(
            [jnp.broadcast_to(esl[h:h + 1, c * _ROWS:(c + 1) * _ROWS],
                              (_ROWS, _ROWS)) for c in range(_CH)], axis=0)
        t = s + esb
        e = jnp.maximum(t, 0.2 * t) + mask
        p = jnp.exp(e)
        dn = jnp.dot(p, ones_h, preferred_element_type=f32)
        part = jnp.concatenate(
            [jnp.dot(p[c * _ROWS:(c + 1) * _ROWS, :],
                     hf[c * _ROWS:(c + 1) * _ROWS, :],
                     preferred_element_type=f32)[:, h * _H:(h + 1) * _H]
             for c in range(_CH)], axis=0)
        part = part / dn
        acc = part if acc is None else acc + part
    return acc * (1.0 / _HEADS) + bias


def _fused_kernel(x_ref, adj_ref, memb_ref, w_ref, c_ref, out_ref):
    f32 = jnp.float32
    x = x_ref[...]
    adjc = adj_ref[...]
    membc = memb_ref[...]

    w1 = w_ref[_R_W1:_R_W1 + _FIN, :]
    w2 = w_ref[_R_W2:_R_W2 + _H, :]
    ad1 = w_ref[_R_AD1:_R_AD1 + _HEADS, :]
    as1 = w_ref[_R_AS1:_R_AS1 + _HEADS, :]
    ad2 = w_ref[_R_AD2:_R_AD2 + _HEADS, :]
    as2 = w_ref[_R_AS2:_R_AS2 + _HEADS, :]
    b1 = w_ref[_R_GB:_R_GB + 1, 0:_H]
    b2 = w_ref[_R_GB + 1:_R_GB + 2, 0:_H]
    wh1_w = w_ref[_R_WH1:_R_WH1 + _H, 0:2 * _H]
    wh1_b = w_ref[_R_WH1B:_R_WH1B + 1, 0:2 * _H]
    wh2_wt = w_ref[_R_WH2:_R_WH2 + 2, 0:2 * _H]
    wh2_b = w_ref[_R_WH2B:_R_WH2B + 1, 0:2]

    same1 = c_ref[_C_SAME:_C_SAME + _ROWS, :]
    psame1 = c_ref[_C_PSAME:_C_PSAME + _PROWS, :]
    grp = c_ref[_C_GRP:_C_GRP + _LANES, :]
    tile = c_ref[_C_TILE:_C_TILE + _S, :]
    sel = [c_ref[_C_SEL + h * _HEADS:_C_SEL + h * _HEADS + _HEADS, :]
           for h in range(_HEADS)]
    same = jnp.concatenate([same1] * _CH, axis=0)
    psame = jnp.concatenate([psame1] * _CH, axis=0)
    ones_h = jnp.ones((_ROWS, _H), f32)
    ones_sh = jnp.ones((_S, _H), f32)

    adj_t = jnp.dot(adjc, tile, preferred_element_type=f32)
    mask = jnp.where(adj_t * same > 0.5, 0.0, -1e9)

    h1 = jnp.maximum(_gat_layer(x, w1, ad1, as1, b1, mask, sel, ones_h), 0.0)
    h2 = jnp.maximum(_gat_layer(h1, w2, ad2, as2, b2, mask, sel, ones_h), 0.0)

    memb_t = jnp.dot(membc, tile, preferred_element_type=f32)
    memb_bd = memb_t * psame
    cntb = jnp.maximum(jnp.dot(membc, ones_sh, preferred_element_type=f32), 1.0)
    sp = jnp.concatenate(
        [jnp.dot(memb_bd[c * _PROWS:(c + 1) * _PROWS, :],
                 h2[c * _ROWS:(c + 1) * _ROWS, :], preferred_element_type=f32)
         for c in range(_CH)], axis=0) / cntb

    hid = jnp.maximum(jnp.dot(sp, wh1_w, preferred_element_type=f32) + wh1_b, 0.0)
    lv = jax.lax.dot_general(hid, wh2_wt,
                             dimension_numbers=(((1,), (1,)), ((), ())),
                             preferred_element_type=f32) + wh2_b

    lane = jax.lax.broadcasted_iota(jnp.int32, (_LANES, 8), 1)
    outs = []
    for q in range(_TPROWS // _LANES):
        lvq = lv[q * _LANES:(q + 1) * _LANES, :]
        logits_c = lvq[:, 0:1]
        values_c = lvq[:, 1:2]
        ez = jnp.exp(logits_c)
        ssum = jnp.dot(grp, ez, preferred_element_type=f32)
        logp = logits_c - jnp.log(ssum)
        pr = jnp.exp(logp)
        ent = -jnp.dot(grp, pr * logp, preferred_element_type=f32)
        outs.append(jnp.where(lane == 0, logits_c, 0.0)
                    + jnp.where(lane == 1, values_c, 0.0)
                    + jnp.where(lane == 2, logp, 0.0)
                    + jnp.where(lane == 3, ent, 0.0))
    out_ref[...] = jnp.concatenate(outs, axis=0).astype(out_ref.dtype)


def kernel(x, adj, memb, w_slab, sample_key):
    n = x.shape[0]
    pairs_per_step = _G * _CH
    nsteps = -(-n // pairs_per_step)
    pad = nsteps * pairs_per_step - n
    if pad:
        x = jnp.concatenate([x, jnp.zeros((pad,) + x.shape[1:], x.dtype)], 0)
        eye = jnp.broadcast_to(jnp.eye(_S, dtype=adj.dtype), (pad, _B, _S, _S))
        adj = jnp.concatenate([adj, eye], 0)
        memb = jnp.concatenate([memb, jnp.zeros((pad,) + memb.shape[1:], memb.dtype)], 0)
    np_ = nsteps * pairs_per_step

    xs = x.reshape(np_ * _B * _S, _FIN)
    adjc = adj.reshape(np_ * _B * _S, _S)
    membc = memb.reshape(np_ * _B * _I, _S)
    cslab = _make_const_slab()

    out = pl.pallas_call(
        _fused_kernel,
        out_shape=jax.ShapeDtypeStruct((np_ * _B * _I, 8), jnp.float32),
        grid=(nsteps,),
        in_specs=[
            pl.BlockSpec((_TROWS, _FIN), lambda g: (g, 0)),
            pl.BlockSpec((_TROWS, _S), lambda g: (g, 0)),
            pl.BlockSpec((_TPROWS, _S), lambda g: (g, 0)),
            pl.BlockSpec((_WBLK, _LANES), lambda g: (0, 0)),
            pl.BlockSpec((_CROWS, _LANES), lambda g: (0, 0)),
        ],
        out_specs=pl.BlockSpec((_TPROWS, 8), lambda g: (g, 0)),
        compiler_params=pltpu.CompilerParams(dimension_semantics=("parallel",)),
        cost_estimate=pl.CostEstimate(flops=int(n * 4.2e6),
                                      transcendentals=int(n * 140_000),
                                      bytes_accessed=int(n * 4200)),
    )(xs, adjc, membc, w_slab, cslab)

    r = out.reshape(np_, _B, _I, 8)[:n]
    logits = r[:, :, :, 0]
    values = r[:, :, :, 1]
    logp_all = r[:, :, :, 2]
    entropy = r[:, :, 0, 3]

    base = jax.random.key(sample_key[0])
    keys = jax.random.split(base, n)
    actions = jax.vmap(lambda ki, lg: jax.random.categorical(ki, lg, axis=-1))(keys, logits)
    log_probs = jnp.take_along_axis(logp_all, actions[:, :, None], axis=-1)[:, :, 0]
    return actions, log_probs, entropy, values


# speedup vs baseline: 1.7007x; 1.1709x over previous
import jax
import jax.numpy as jnp
from jax.experimental import pallas as pl
from jax.experimental.pallas import tpu as pltpu

_B = 2
_S = 16
_I = 4
_FIN = 8
_H = 32
_HEADS = 4
_G = 4
_ROWS = _G * _B * _S
_PROWS = _G * _B * _I
_CH = 16
_TROWS = _CH * _ROWS
_TPROWS = _CH * _PROWS
_LANES = 128

_R_W1 = 0
_R_W2 = 8
_R_AD1 = 40
_R_AS1 = 48
_R_AD2 = 56
_R_AS2 = 64
_R_GB = 72
_R_WH1 = 80
_R_WH1B = 112
_R_WH2 = 120
_R_WH2B = 128
_WBLK = 136

_C_SAME = 0
_C_PSAME = 128
_C_GRP = 160
_C_TILE = 288
_C_SEL = 304
_CROWS = 320


def _make_const_slab():
    f32 = jnp.float32
    idx = jnp.arange(_ROWS)
    same = (idx[:, None] // _S == idx[None, :] // _S).astype(f32)
    pidx = jnp.arange(_PROWS)
    psame = (pidx[:, None] // _I == idx[None, :] // _S).astype(f32)
    tp = jnp.arange(_LANES)
    grp = (tp[:, None] // _I == tp[None, :] // _I).astype(f32)
    tile = (jnp.arange(_LANES)[None, :] % _S == jnp.arange(_S)[:, None]).astype(f32)
    slab = jnp.zeros((_CROWS, _LANES), f32)
    slab = slab.at[_C_SAME:_C_SAME + _ROWS, :].set(same)
    slab = slab.at[_C_PSAME:_C_PSAME + _PROWS, :].set(psame)
    slab = slab.at[_C_GRP:_C_GRP + _LANES, :].set(grp)
    slab = slab.at[_C_TILE:_C_TILE + _S, :].set(tile)
    sel_rows = jnp.zeros((_HEADS * _HEADS, _LANES), f32)
    for h in range(_HEADS):
        sel_rows = sel_rows.at[h * _HEADS + h, :].set(1.0)
    slab = slab.at[_C_SEL:_C_SEL + _HEADS * _HEADS, :].set(sel_rows)
    return slab


def _gat_layer(feat, w_all, a_dst_rows, a_src_rows, bias, mask, sel, ones_h):
    f32 = jnp.float32
    hf = jnp.dot(feat, w_all, preferred_element_type=f32)
    ed = jax.lax.dot_general(hf, a_dst_rows,
                             dimension_numbers=(((1,), (1,)), ((), ())),
                             preferred_element_type=f32)
    esl = jax.lax.dot_general(a_src_rows, hf,
                              dimension_numbers=(((1,), (1,)), ((), ())),
                              preferred_element_type=f32)

    acc = None
    for h in range(_HEADS):
        s = jnp.dot(ed, sel[h], preferred_element_type=f32)
        esb = jnp.concatenate(
            [jnp.broadcast_to(esl[h:h + 1, c * _ROWS:(c + 1) * _ROWS],
                              (_ROWS, _ROWS)) for c in range(_CH)], axis=0)
        t = s + esb
        e = jnp.maximum(t, 0.2 * t) + mask
        p = jnp.exp(e)
        dn = jnp.dot(p, ones_h, preferred_element_type=f32)
        part = jnp.concatenate(
            [jnp.dot(p[c * _ROWS:(c + 1) * _ROWS, :],
                     hf[c * _ROWS:(c + 1) * _ROWS, :],
                     preferred_element_type=f32)[:, h * _H:(h + 1) * _H]
             for c in range(_CH)], axis=0)
        part = part / dn
        acc = part if acc is None else acc + part
    return acc * (1.0 / _HEADS) + bias


def _fused_kernel(x_ref, adj_ref, memb_ref, w_ref, c_ref, out_ref):
    f32 = jnp.float32
    x = x_ref[...]
    adjc = adj_ref[...]
    membc = memb_ref[...]

    w1 = w_ref[_R_W1:_R_W1 + _FIN, :]
    w2 = w_ref[_R_W2:_R_W2 + _H, :]
    ad1 = w_ref[_R_AD1:_R_AD1 + _HEADS, :]
    as1 = w_ref[_R_AS1:_R_AS1 + _HEADS, :]
    ad2 = w_ref[_R_AD2:_R_AD2 + _HEADS, :]
    as2 = w_ref[_R_AS2:_R_AS2 + _HEADS, :]
    b1 = w_ref[_R_GB:_R_GB + 1, 0:_H]
    b2 = w_ref[_R_GB + 1:_R_GB + 2, 0:_H]
    wh1_w = w_ref[_R_WH1:_R_WH1 + _H, 0:2 * _H]
    wh1_b = w_ref[_R_WH1B:_R_WH1B + 1, 0:2 * _H]
    wh2_wt = w_ref[_R_WH2:_R_WH2 + 2, 0:2 * _H]
    wh2_b = w_ref[_R_WH2B:_R_WH2B + 1, 0:2]

    same1 = c_ref[_C_SAME:_C_SAME + _ROWS, :]
    psame1 = c_ref[_C_PSAME:_C_PSAME + _PROWS, :]
    grp = c_ref[_C_GRP:_C_GRP + _LANES, :]
    tile = c_ref[_C_TILE:_C_TILE + _S, :]
    sel = [c_ref[_C_SEL + h * _HEADS:_C_SEL + h * _HEADS + _HEADS, :]
           for h in range(_HEADS)]
    same = jnp.concatenate([same1] * _CH, axis=0)
    psame = jnp.concatenate([psame1] * _CH, axis=0)
    ones_h = jnp.ones((_ROWS, _H), f32)
    ones_sh = jnp.ones((_S, _H), f32)

    adj_t = jnp.dot(adjc, tile, preferred_element_type=f32)
    mask = jnp.where(adj_t * same > 0.5, 0.0, -1e9)

    h1 = jnp.maximum(_gat_layer(x, w1, ad1, as1, b1, mask, sel, ones_h), 0.0)
    h2 = jnp.maximum(_gat_layer(h1, w2, ad2, as2, b2, mask, sel, ones_h), 0.0)

    memb_t = jnp.dot(membc, tile, preferred_element_type=f32)
    memb_bd = memb_t * psame
    cntb = jnp.maximum(jnp.dot(membc, ones_sh, preferred_element_type=f32), 1.0)
    sp = jnp.concatenate(
        [jnp.dot(memb_bd[c * _PROWS:(c + 1) * _PROWS, :],
                 h2[c * _ROWS:(c + 1) * _ROWS, :], preferred_element_type=f32)
         for c in range(_CH)], axis=0) / cntb

    hid = jnp.maximum(jnp.dot(sp, wh1_w, preferred_element_type=f32) + wh1_b, 0.0)
    lv = jax.lax.dot_general(hid, wh2_wt,
                             dimension_numbers=(((1,), (1,)), ((), ())),
                             preferred_element_type=f32) + wh2_b

    lane = jax.lax.broadcasted_iota(jnp.int32, (_LANES, 8), 1)
    outs = []
    for q in range(_TPROWS // _LANES):
        lvq = lv[q * _LANES:(q + 1) * _LANES, :]
        logits_c = lvq[:, 0:1]
        values_c = lvq[:, 1:2]
        ez = jnp.exp(logits_c)
        ssum = jnp.dot(grp, ez, preferred_element_type=f32)
        logp = logits_c - jnp.log(ssum)
        pr = jnp.exp(logp)
        ent = -jnp.dot(grp, pr * logp, preferred_element_type=f32)
        outs.append(jnp.where(lane == 0, logits_c, 0.0)
                    + jnp.where(lane == 1, values_c, 0.0)
                    + jnp.where(lane == 2, logp, 0.0)
                    + jnp.where(lane == 3, ent, 0.0))
    out_ref[...] = jnp.concatenate(outs, axis=0).astype(out_ref.dtype)


def kernel(x, adj, memb, w_slab, sample_key):
    n = x.shape[0]
    pairs_per_step = _G * _CH
    nsteps = -(-n // pairs_per_step)
    pad = nsteps * pairs_per_step - n
    if pad:
        x = jnp.concatenate([x, jnp.zeros((pad,) + x.shape[1:], x.dtype)], 0)
        eye = jnp.broadcast_to(jnp.eye(_S, dtype=adj.dtype), (pad, _B, _S, _S))
        adj = jnp.concatenate([adj, eye], 0)
        memb = jnp.concatenate([memb, jnp.zeros((pad,) + memb.shape[1:], memb.dtype)], 0)
    np_ = nsteps * pairs_per_step

    xs = x.reshape(np_ * _B * _S, _FIN)
    adjc = adj.reshape(np_ * _B * _S, _S)
    membc = memb.reshape(np_ * _B * _I, _S)
    cslab = _make_const_slab()

    out = pl.pallas_call(
        _fused_kernel,
        out_shape=jax.ShapeDtypeStruct((np_ * _B * _I, 8), jnp.float32),
        grid=(nsteps,),
        in_specs=[
            pl.BlockSpec((_TROWS, _FIN), lambda g: (g, 0)),
            pl.BlockSpec((_TROWS, _S), lambda g: (g, 0)),
            pl.BlockSpec((_TPROWS, _S), lambda g: (g, 0)),
            pl.BlockSpec((_WBLK, _LANES), lambda g: (0, 0)),
            pl.BlockSpec((_CROWS, _LANES), lambda g: (0, 0)),
        ],
        out_specs=pl.BlockSpec((_TPROWS, 8), lambda g: (g, 0)),
        compiler_params=pltpu.CompilerParams(dimension_semantics=("parallel",)),
        cost_estimate=pl.CostEstimate(flops=int(n * 4.2e6),
                                      transcendentals=int(n * 140_000),
                                      bytes_accessed=int(n * 4200)),
    )(xs, adjc, membc, w_slab, cslab)

    r = out.reshape(np_, _B, _I, 8)[:n]
    logits = r[:, :, :, 0]
    values = r[:, :, :, 1]
    logp_all = r[:, :, :, 2]
    entropy = r[:, :, 0, 3]

    base = jax.random.key(sample_key[0])
    keys = jax.random.split(base, n)
    actions = jax.vmap(lambda ki, lg: jax.random.categorical(ki, lg, axis=-1))(keys, logits)
    log_probs = jnp.take_along_axis(logp_all, actions[:, :, None], axis=-1)[:, :, 0]
    return actions, log_probs, entropy, values
